# baseline (speedup 1.0000x reference)
import functools

import jax
import jax.numpy as jnp
from jax import lax
from jax.experimental import pallas as pl
from jax.experimental.pallas import tpu as pltpu
from jax.experimental.pallas import tpu_sc as plsc

B = 16384
DP, DC, DS = 32, 16, 24
PV = 100001
PV_PAD = 100008
PTW = 48
DO = DP + DC + DS
NC, NS = 2, 16
NW = NC * NS
BW = B // NW

_mesh = plsc.VectorSubcoreMesh(core_axis_name="c", subcore_axis_name="s")


DOP = 128

@functools.partial(
    pl.kernel,
    out_type=jax.ShapeDtypeStruct((B, DOP), jnp.float32),
    mesh=_mesh,
    scratch_types=[
        pltpu.VMEM((BW,), jnp.int32),
        pltpu.VMEM((BW,), jnp.int32),
        pltpu.VMEM((BW,), jnp.int32),
        pltpu.VMEM((BW, PTW), jnp.float32),
        pltpu.VMEM((BW, DC), jnp.float32),
        pltpu.VMEM((BW, DS), jnp.float32),
        pltpu.VMEM((BW, DOP), jnp.float32),
        pltpu.SemaphoreType.DMA,
    ],
    compiler_params=pltpu.CompilerParams(use_tc_tiling_on_sc=False),
)
def _sc_kernel(pid_hbm, cid_hbm, sid_hbm, ptab_hbm, ctab_hbm, stab_hbm,
               out_hbm, pidx_v, cidx_v, sidx_v, prod_v, cat_v, sub_v,
               row_v, sem):
    wid = lax.axis_index("s") * NC + lax.axis_index("c")
    base = wid * BW
    pltpu.sync_copy(pid_hbm.at[pl.ds(base, BW)], pidx_v)
    pltpu.sync_copy(cid_hbm.at[pl.ds(base, BW)], cidx_v)
    pltpu.sync_copy(sid_hbm.at[pl.ds(base, BW)], sidx_v)
    cp1 = pltpu.async_copy(ptab_hbm.at[pidx_v], prod_v, sem)
    cp2 = pltpu.async_copy(ctab_hbm.at[cidx_v], cat_v, sem)
    cp3 = pltpu.async_copy(stab_hbm.at[sidx_v], sub_v, sem)
    cp1.wait()
    cp2.wait()
    cp3.wait()

    @plsc.parallel_loop(0, BW, unroll=8)
    def _assemble(r):
        row_v[r, pl.ds(0, 16)] = prod_v[r, pl.ds(0, 16)]
        row_v[r, pl.ds(16, 16)] = prod_v[r, pl.ds(16, 16)]
        row_v[r, pl.ds(32, 16)] = cat_v[r, pl.ds(0, 16)]
        row_v[r, pl.ds(48, 16)] = sub_v[r, pl.ds(0, 16)]
        row_v[r, pl.ds(56, 16)] = sub_v[r, pl.ds(8, 16)]

    pltpu.sync_copy(row_v, out_hbm.at[pl.ds(base, BW)])


def kernel(product_id, stratbuy_domain_desc, mge_main_cat_desc,
           product_table, category_table, subcategory_table):
    ptab128 = jnp.pad(product_table, ((0, PV_PAD - PV), (0, PTW - DP)))
    out = _sc_kernel(
        product_id.astype(jnp.int32),
        stratbuy_domain_desc.astype(jnp.int32),
        mge_main_cat_desc.astype(jnp.int32),
        ptab128, category_table, subcategory_table)
    return out[:, :DO]


# speedup vs baseline: 1.4941x; 1.4941x over previous
import functools

import jax
import jax.numpy as jnp
from jax import lax
from jax.experimental import pallas as pl
from jax.experimental.pallas import tpu as pltpu
from jax.experimental.pallas import tpu_sc as plsc

B = 16384
DP, DC, DS = 32, 16, 24
PV = 100001
PV_PAD = 100008
DO = DP + DC + DS
NC, NS = 2, 16
NW = NC * NS
BW = B // NW

_mesh = plsc.VectorSubcoreMesh(core_axis_name="c", subcore_axis_name="s")


DOP = 128

@functools.partial(
    pl.kernel,
    out_type=jax.ShapeDtypeStruct((B, DOP), jnp.float32),
    mesh=_mesh,
    scratch_types=[
        pltpu.VMEM((BW,), jnp.int32),
        pltpu.VMEM((BW,), jnp.int32),
        pltpu.VMEM((BW,), jnp.int32),
        pltpu.VMEM((BW, DC), jnp.float32),
        pltpu.VMEM((BW, DS), jnp.float32),
        pltpu.VMEM((BW, DOP), jnp.float32),
        pltpu.SemaphoreType.DMA,
    ],
    compiler_params=pltpu.CompilerParams(use_tc_tiling_on_sc=False),
)
def _sc_kernel(pid_hbm, cid_hbm, sid_hbm, ptab_hbm, ctab_hbm, stab_hbm,
               out_hbm, pidx_v, cidx_v, sidx_v, cat_v, sub_v,
               row_v, sem):
    wid = lax.axis_index("s") * NC + lax.axis_index("c")
    base = wid * BW
    pltpu.sync_copy(pid_hbm.at[pl.ds(base, BW)], pidx_v)
    pltpu.sync_copy(cid_hbm.at[pl.ds(base, BW)], cidx_v)
    pltpu.sync_copy(sid_hbm.at[pl.ds(base, BW)], sidx_v)
    cp1 = pltpu.async_copy(ptab_hbm.at[pidx_v], row_v, sem)
    cp2 = pltpu.async_copy(ctab_hbm.at[cidx_v], cat_v, sem)
    cp3 = pltpu.async_copy(stab_hbm.at[sidx_v], sub_v, sem)
    cp1.wait()
    cp2.wait()
    cp3.wait()

    @plsc.parallel_loop(0, BW, unroll=8)
    def _assemble(r):
        row_v[r, pl.ds(32, 16)] = cat_v[r, pl.ds(0, 16)]
        row_v[r, pl.ds(48, 16)] = sub_v[r, pl.ds(0, 16)]
        row_v[r, pl.ds(56, 16)] = sub_v[r, pl.ds(8, 16)]

    pltpu.sync_copy(row_v, out_hbm.at[pl.ds(base, BW)])


def kernel(product_id, stratbuy_domain_desc, mge_main_cat_desc,
           product_table, category_table, subcategory_table):
    ptab128 = jnp.pad(product_table, ((0, PV_PAD - PV), (0, DOP - DP)))
    out = _sc_kernel(
        product_id.astype(jnp.int32),
        stratbuy_domain_desc.astype(jnp.int32),
        mge_main_cat_desc.astype(jnp.int32),
        ptab128, category_table, subcategory_table)
    return out[:, :DO]


# speedup vs baseline: 1.4995x; 1.0036x over previous
import functools

import jax
import jax.numpy as jnp
from jax import lax
from jax.experimental import pallas as pl
from jax.experimental.pallas import tpu as pltpu
from jax.experimental.pallas import tpu_sc as plsc

B = 16384
DP, DC, DS = 32, 16, 24
PV = 100001
PV_PAD = 100008
DO = DP + DC + DS
NC, NS = 2, 16
NW = NC * NS
BW = B // NW

_mesh = plsc.VectorSubcoreMesh(core_axis_name="c", subcore_axis_name="s")


DOP = 128

@functools.partial(
    pl.kernel,
    out_type=jax.ShapeDtypeStruct((B, DOP), jnp.float32),
    mesh=_mesh,
    scratch_types=[
        pltpu.VMEM((BW,), jnp.int32),
        pltpu.VMEM((BW,), jnp.int32),
        pltpu.VMEM((BW,), jnp.int32),
        pltpu.VMEM((BW, DC), jnp.float32),
        pltpu.VMEM((BW, DS), jnp.float32),
        pltpu.VMEM((BW, DOP), jnp.float32),
        [pltpu.SemaphoreType.DMA] * 4,
        pltpu.SemaphoreType.DMA,
    ],
    compiler_params=pltpu.CompilerParams(use_tc_tiling_on_sc=False),
)
def _sc_kernel(pid_hbm, cid_hbm, sid_hbm, ptab_hbm, ctab_hbm, stab_hbm,
               out_hbm, pidx_v, cidx_v, sidx_v, cat_v, sub_v,
               row_v, sems, sem_out):
    wid = lax.axis_index("s") * NC + lax.axis_index("c")
    base = wid * BW
    pltpu.sync_copy(pid_hbm.at[pl.ds(base, BW)], pidx_v)
    pltpu.sync_copy(cid_hbm.at[pl.ds(base, BW)], cidx_v)
    pltpu.sync_copy(sid_hbm.at[pl.ds(base, BW)], sidx_v)
    CH, CB = 4, BW // 4
    gathers = []
    for c in range(CH):
        s = sems[c]
        lo = c * CB
        gathers.append((
            pltpu.async_copy(ptab_hbm.at[pidx_v.at[pl.ds(lo, CB)]],
                             row_v.at[pl.ds(lo, CB), :], s),
            pltpu.async_copy(ctab_hbm.at[cidx_v.at[pl.ds(lo, CB)]],
                             cat_v.at[pl.ds(lo, CB), :], s),
            pltpu.async_copy(stab_hbm.at[sidx_v.at[pl.ds(lo, CB)]],
                             sub_v.at[pl.ds(lo, CB), :], s),
        ))
    outs = []
    for c in range(CH):
        for g in gathers[c]:
            g.wait()
        lo = c * CB

        @plsc.parallel_loop(lo, lo + CB, unroll=8)
        def _assemble(r):
            row_v[r, pl.ds(32, 16)] = cat_v[r, pl.ds(0, 16)]
            row_v[r, pl.ds(48, 16)] = sub_v[r, pl.ds(0, 16)]
            row_v[r, pl.ds(56, 16)] = sub_v[r, pl.ds(8, 16)]

        outs.append(pltpu.async_copy(row_v.at[pl.ds(lo, CB), :],
                                     out_hbm.at[pl.ds(base + lo, CB)],
                                     sem_out))
    for o in outs:
        o.wait()

def kernel(product_id, stratbuy_domain_desc, mge_main_cat_desc,
           product_table, category_table, subcategory_table):
    ptab128 = jnp.pad(product_table, ((0, PV_PAD - PV), (0, DOP - DP)))
    out = _sc_kernel(
        product_id.astype(jnp.int32),
        stratbuy_domain_desc.astype(jnp.int32),
        mge_main_cat_desc.astype(jnp.int32),
        ptab128, category_table, subcategory_table)
    return out[:, :DO]
